# baseline (speedup 1.0000x reference)
import functools

import jax
import jax.numpy as jnp
from jax import lax
from jax.experimental import pallas as pl
from jax.experimental.pallas import tpu as pltpu
from jax.experimental.pallas import tpu_sc as plsc

HIDDEN = 64
SCALE = 8.0

NC = 2
NS = 16
NW = NC * NS

C = 128
LANES = 16
PITCH = HIDDEN + 1

REPACK_ROWS = 2000


def _repack(w):
    v, h = w.shape
    assert h == HIDDEN and v % (2 * REPACK_ROWS) == 0
    grid = v // REPACK_ROWS

    def body(x_ref, o_ref):
        x3 = x_ref[...].reshape(REPACK_ROWS // 2, 2, HIDDEN)
        o_ref[:, 0:HIDDEN] = x3[:, 0, :]
        o_ref[:, HIDDEN:2 * HIDDEN] = x3[:, 1, :]

    return pl.pallas_call(
        body,
        grid=(grid,),
        in_specs=[pl.BlockSpec((REPACK_ROWS, HIDDEN), lambda i: (i, 0))],
        out_specs=pl.BlockSpec((REPACK_ROWS // 2, 2 * HIDDEN), lambda i: (i, 0)),
        out_shape=jax.ShapeDtypeStruct((v // 2, 2 * HIDDEN), jnp.float32),
    )(w)


def _make_emb_kernel(S, B):
    total = S * B
    assert B % C == 0 and total % NW == 0
    bpw = total // NW
    assert bpw % C == 0
    nchunk = bpw // C
    assert nchunk % 2 == 0

    mesh = plsc.VectorSubcoreMesh(core_axis_name="c", subcore_axis_name="s")

    @functools.partial(
        pl.kernel,
        mesh=mesh,
        out_type=jax.ShapeDtypeStruct((S, HIDDEN // 8, B // C, 8 * C), jnp.float32),
        compiler_params=pltpu.CompilerParams(
            use_tc_tiling_on_sc=False, needs_layout_passes=False),
        scratch_types=[
            pltpu.VMEM((bpw,), jnp.int32),
            pltpu.VMEM((C, HIDDEN), jnp.float32),
            pltpu.VMEM((C, HIDDEN), jnp.float32),
            pltpu.VMEM((C * PITCH,), jnp.float32),
            pltpu.VMEM((HIDDEN // 8, 8 * C), jnp.float32),
            pltpu.VMEM((HIDDEN // 8, 8 * C), jnp.float32),
            pltpu.SemaphoreType.DMA,
            pltpu.SemaphoreType.DMA,
        ],
    )
    def emb(idx_hbm, tab_hbm, out_hbm, idx_v, rows0, rows1, rpad,
            obuf0, obuf1, sem0, sem1):
        wid = lax.axis_index("s") * NC + lax.axis_index("c")
        base = wid * bpw
        sems = (sem0, sem1)
        rows = (rows0, rows1)
        obufs = (obuf0, obuf1)

        pltpu.sync_copy(idx_hbm.at[pl.ds(base, bpw)], idx_v)

        riota = lax.broadcasted_iota(jnp.int32, (LANES,), 0)
        rbase = [(riota + b0 * LANES) * PITCH for b0 in range(C // LANES)]

        def start(g, slot):
            pltpu.async_copy(
                tab_hbm.at[idx_v.at[pl.ds(g * C, C)]],
                rows[slot],
                sems[slot],
            )

        def wait(g, slot):
            pltpu.make_async_copy(
                tab_hbm.at[idx_v.at[pl.ds(g * C, C)]],
                rows[slot],
                sems[slot],
            ).wait()

        def transform(slot):
            @plsc.parallel_loop(0, C, step=4, unroll=2)
            def pbody(b):
                for bb in range(4):
                    for k in range(HIDDEN // LANES):
                        v = rows[slot][b + bb, pl.ds(k * LANES, LANES)]
                        rpad[pl.ds((b + bb) * PITCH + k * LANES, LANES)] = v * SCALE

            @plsc.parallel_loop(0, HIDDEN // 8, step=1, unroll=2)
            def hbody(hq):
                for hr in range(8):
                    h = hq * 8 + hr
                    for b0 in range(C // LANES):
                        v = plsc.load_gather(rpad, [rbase[b0] + h])
                        obufs[slot][hq, pl.ds(hr * C + b0 * LANES, LANES)] = v

        def finish(g, slot):
            wait(g, slot)
            transform(slot)
            t0 = base + g * C
            s = t0 // B
            bb = (t0 % B) // C
            pltpu.sync_copy(obufs[slot], out_hbm.at[s, :, bb])

        start(0, 0)

        def pair(p, carry):
            g0 = 2 * p
            start(g0 + 1, 1)
            finish(g0, 0)
            start(g0 + 2, 0)
            finish(g0 + 1, 1)
            return carry

        lax.fori_loop(0, nchunk // 2 - 1, pair, 0)

        g0 = nchunk - 2
        start(g0 + 1, 1)
        finish(g0, 0)
        finish(g0 + 1, 1)

    return emb


def kernel(x, weights):
    b, s = x.shape
    v, hidden = weights.shape
    xf = x.T.reshape(-1).astype(jnp.int32)
    w3 = _repack(weights).reshape(v, hidden)
    out5 = _make_emb_kernel(s, b)(xf, w3)
    out5 = out5.reshape(s, HIDDEN // 8, b // C, 8, C)
    return out5.transpose(2, 4, 0, 1, 3).reshape(b, s, HIDDEN)
